# baseline (speedup 1.0000x reference)
import functools

import jax
import jax.numpy as jnp
from jax import lax
from jax.experimental import pallas as pl
from jax.experimental.pallas import tpu as pltpu
from jax.experimental.pallas import tpu_sc as plsc

N = 10000
E = 320000
F_IN = 128
H = 16
OUT = 10

NC = 2
NS = 16
NW = NC * NS
NPAD = 10240
SL = NPAD // NS
CHUNK = 128
G = 8
GS = 8
EP = 327680
RW = EP // (NW * CHUNK)

_mesh = plsc.VectorSubcoreMesh(core_axis_name="c", subcore_axis_name="s")


@functools.partial(
    pl.kernel,
    mesh=_mesh,
    out_type=jax.ShapeDtypeStruct((NC, NPAD), jnp.float32),
    scratch_types=[
        pltpu.VMEM((G * CHUNK,), jnp.int32),
        pltpu.VMEM((CHUNK,), jnp.int32),
        pltpu.VMEM((CHUNK,), jnp.float32),
        pltpu.VMEM_SHARED((NPAD,), jnp.float32),
    ],
)
def _deg_kernel(col_hbm, zeros_hbm, out_hbm, idx_v, cjdx_v, ones_v, deg_sh):
    c = lax.axis_index("c")
    s = lax.axis_index("s")
    w = c * NS + s
    for k in range(CHUNK // 16):
        ones_v[pl.ds(k * 16, 16)] = jnp.ones((16,), jnp.float32)
    pltpu.sync_copy(zeros_hbm.at[pl.ds(s * SL, SL)], deg_sh.at[pl.ds(s * SL, SL)])
    plsc.subcore_barrier()

    def outer(i, carry):
        base = (w * RW + i * G) * CHUNK
        pltpu.sync_copy(col_hbm.at[pl.ds(base, G * CHUNK)], idx_v)
        for j in range(G):
            for k in range(CHUNK // 16):
                cjdx_v[pl.ds(k * 16, 16)] = idx_v[pl.ds(j * CHUNK + k * 16, 16)]
            pltpu.sync_copy(ones_v, deg_sh.at[cjdx_v], add=True)
        return carry

    lax.fori_loop(0, RW // G, outer, 0)
    plsc.subcore_barrier()
    pltpu.sync_copy(deg_sh.at[pl.ds(s * SL, SL)], out_hbm.at[c, pl.ds(s * SL, SL)])


@functools.partial(
    pl.kernel,
    mesh=_mesh,
    out_type=(
        jax.ShapeDtypeStruct((NC, NPAD, H), jnp.float32),
        jax.ShapeDtypeStruct((NC, NPAD), jnp.float32),
    ),
    scratch_types=[
        pltpu.VMEM((GS * CHUNK,), jnp.int32),
        pltpu.VMEM((GS * CHUNK,), jnp.int32),
        pltpu.VMEM((CHUNK,), jnp.int32),
        pltpu.VMEM((CHUNK,), jnp.int32),
        pltpu.VMEM((GS * CHUNK, H), jnp.float32),
        pltpu.VMEM((GS * CHUNK,), jnp.float32),
        pltpu.VMEM_SHARED((NPAD, H), jnp.float32),
        pltpu.VMEM_SHARED((NPAD,), jnp.float32),
        pltpu.SemaphoreType.DMA,
        pltpu.SemaphoreType.DMA,
    ],
    compiler_params=pltpu.CompilerParams(use_tc_tiling_on_sc=False),
)
def _scatter_kernel(row1_hbm, col1_hbm, y_hbm, dinv_hbm,
                    z1_hbm, z2_hbm, acc_out, t_out,
                    ridx1_v, cidx1_v, rjdx_v, cjdx_v, rows_v, dv_v,
                    acc_sh, t_sh, sem, sem2):
    c = lax.axis_index("c")
    s = lax.axis_index("s")
    w = c * NS + s
    pltpu.sync_copy(z2_hbm.at[pl.ds(s * SL, SL)], acc_sh.at[pl.ds(s * SL, SL)])
    pltpu.sync_copy(z1_hbm.at[pl.ds(s * SL, SL)], t_sh.at[pl.ds(s * SL, SL)])
    plsc.subcore_barrier()

    def outer(i, carry):
        base = (w * RW + i * GS) * CHUNK
        pltpu.sync_copy(row1_hbm.at[pl.ds(base, GS * CHUNK)], ridx1_v)
        pltpu.sync_copy(col1_hbm.at[pl.ds(base, GS * CHUNK)], cidx1_v)
        cp1 = pltpu.async_copy(y_hbm.at[ridx1_v], rows_v, sem)
        cp2 = pltpu.async_copy(dinv_hbm.at[cidx1_v], dv_v, sem2)
        cp1.wait()
        cp2.wait()
        for j in range(GS):
            for k in range(CHUNK // 16):
                o = j * CHUNK + k * 16
                cjdx_v[pl.ds(k * 16, 16)] = cidx1_v[pl.ds(o, 16)]
                rjdx_v[pl.ds(k * 16, 16)] = ridx1_v[pl.ds(o, 16)]
            pltpu.sync_copy(rows_v.at[pl.ds(j * CHUNK, CHUNK)],
                            acc_sh.at[cjdx_v], add=True)
            pltpu.sync_copy(dv_v.at[pl.ds(j * CHUNK, CHUNK)],
                            t_sh.at[rjdx_v], add=True)
        return carry

    lax.fori_loop(0, RW // GS, outer, 0)
    plsc.subcore_barrier()
    pltpu.sync_copy(acc_sh.at[pl.ds(s * SL, SL)], acc_out.at[c, pl.ds(s * SL, SL)])
    pltpu.sync_copy(t_sh.at[pl.ds(s * SL, SL)], t_out.at[c, pl.ds(s * SL, SL)])


def _scale_body(x_ref, w1_ref, degp_ref, y_ref, dinv_ref):
    deg = degp_ref[:, 0:1] + degp_ref[:, 1:2] + 1.0
    dinv = lax.rsqrt(deg)
    dinv_ref[...] = dinv
    xw = jnp.dot(x_ref[...], w1_ref[...], preferred_element_type=jnp.float32)
    y_ref[...] = xw * dinv


_scale_call = pl.pallas_call(
    _scale_body,
    out_shape=(
        jax.ShapeDtypeStruct((NPAD, H), jnp.float32),
        jax.ShapeDtypeStruct((NPAD, 1), jnp.float32),
    ),
)


def _final_body(accp_ref, tp_ref, y_ref, dinv_ref, b1_ref, w2_ref, b2_ref,
                lw_ref, lb_ref, out_ref):
    dinv = dinv_ref[...]
    acc = accp_ref[0] + accp_ref[1]
    h = jnp.maximum(dinv * (acc + y_ref[...]) + b1_ref[...], 0.0)
    t = tp_ref[:, 0:1] + tp_ref[:, 1:2]
    s = dinv * (dinv + t)
    ridx = lax.broadcasted_iota(jnp.int32, (NPAD, 1), 0)
    s = jnp.where(ridx < N, s, 0.0)
    z = jnp.sum(s * h, axis=0, keepdims=True)
    pooled = jnp.dot(z * (1.0 / N), w2_ref[...],
                     preferred_element_type=jnp.float32) + b2_ref[...]
    out_ref[...] = jnp.dot(pooled, lw_ref[...],
                           preferred_element_type=jnp.float32) + lb_ref[...]


_final_call = pl.pallas_call(
    _final_body,
    out_shape=jax.ShapeDtypeStruct((1, OUT), jnp.float32),
)


def kernel(x, edge_index, W1, b1, W2, b2, lin_W, lin_b):
    row = edge_index[0]
    col = edge_index[1]
    npad_extra = NPAD - N
    pad_idx = (N + jnp.arange(EP - E, dtype=jnp.int32) % npad_extra)
    rowp = jnp.concatenate([row, pad_idx])
    colp = jnp.concatenate([col, pad_idx])
    zeros1 = jnp.zeros((NPAD,), jnp.float32)
    zeros2 = jnp.zeros((NPAD, H), jnp.float32)

    degp = _deg_kernel(colp, zeros1)
    xp = jnp.pad(x, ((0, npad_extra), (0, 0)))
    y, dinv2 = _scale_call(xp, W1, degp.T)
    dinv1 = dinv2.reshape(NPAD)
    accp, tp = _scatter_kernel(rowp, colp, y, dinv1, zeros1, zeros2)
    out = _final_call(accp, tp.T, y, dinv2, b1.reshape(1, H), W2,
                      b2.reshape(1, H), lin_W, lin_b.reshape(1, OUT))
    return out.reshape(OUT)


# speedup vs baseline: 1.0895x; 1.0895x over previous
import functools

import jax
import jax.numpy as jnp
from jax import lax
from jax.experimental import pallas as pl
from jax.experimental.pallas import tpu as pltpu
from jax.experimental.pallas import tpu_sc as plsc

N = 10000
E = 320000
F_IN = 128
H = 16
OUT = 10

NC = 2
NS = 16
NW = NC * NS
NPAD = 10240
SL = NPAD // NS
CHUNK = 128
G = 8
GS = 8
EP = 327680
RW = EP // (NW * CHUNK)

_mesh = plsc.VectorSubcoreMesh(core_axis_name="c", subcore_axis_name="s")


@functools.partial(
    pl.kernel,
    mesh=_mesh,
    out_type=jax.ShapeDtypeStruct((NC, NPAD), jnp.float32),
    scratch_types=[
        pltpu.VMEM((G * CHUNK,), jnp.int32),
        pltpu.VMEM((G, CHUNK), jnp.int32),
        pltpu.VMEM((CHUNK,), jnp.float32),
        pltpu.VMEM_SHARED((NPAD,), jnp.float32),
        pltpu.SemaphoreType.DMA,
    ],
)
def _deg_kernel(col_hbm, zeros_hbm, out_hbm, idx_v, cidx2_v, ones_v, deg_sh,
                sem):
    c = lax.axis_index("c")
    s = lax.axis_index("s")
    w = c * NS + s
    for k in range(CHUNK // 16):
        ones_v[pl.ds(k * 16, 16)] = jnp.ones((16,), jnp.float32)
    pltpu.sync_copy(zeros_hbm.at[pl.ds(s * SL, SL)], deg_sh.at[pl.ds(s * SL, SL)])
    plsc.subcore_barrier()

    def outer(i, carry):
        base = (w * RW + i * G) * CHUNK
        pltpu.sync_copy(col_hbm.at[pl.ds(base, G * CHUNK)], idx_v)
        for j in range(G):
            for k in range(CHUNK // 16):
                cidx2_v[j, pl.ds(k * 16, 16)] = idx_v[pl.ds(j * CHUNK + k * 16, 16)]
        cps = []
        for j in range(G):
            cps.append(pltpu.async_copy(ones_v, deg_sh.at[cidx2_v.at[j]],
                                        sem, add=True))
        for cp in cps:
            cp.wait()
        return carry

    lax.fori_loop(0, RW // G, outer, 0)
    plsc.subcore_barrier()
    pltpu.sync_copy(deg_sh.at[pl.ds(s * SL, SL)], out_hbm.at[c, pl.ds(s * SL, SL)])


@functools.partial(
    pl.kernel,
    mesh=_mesh,
    out_type=(
        jax.ShapeDtypeStruct((NC, NPAD, H), jnp.float32),
        jax.ShapeDtypeStruct((NC, NPAD), jnp.float32),
    ),
    scratch_types=[
        pltpu.VMEM((GS * CHUNK,), jnp.int32),
        pltpu.VMEM((GS * CHUNK,), jnp.int32),
        pltpu.VMEM((GS, CHUNK), jnp.int32),
        pltpu.VMEM((GS, CHUNK), jnp.int32),
        pltpu.VMEM((GS * CHUNK, H), jnp.float32),
        pltpu.VMEM((GS * CHUNK,), jnp.float32),
        pltpu.VMEM_SHARED((NPAD, H), jnp.float32),
        pltpu.VMEM_SHARED((NPAD,), jnp.float32),
        pltpu.SemaphoreType.DMA,
        pltpu.SemaphoreType.DMA,
        pltpu.SemaphoreType.DMA,
        pltpu.SemaphoreType.DMA,
    ],
    compiler_params=pltpu.CompilerParams(use_tc_tiling_on_sc=False),
)
def _scatter_kernel(row1_hbm, col1_hbm, y_hbm, dinv_hbm,
                    z1_hbm, z2_hbm, acc_out, t_out,
                    ridx1_v, cidx1_v, ridx2_v, cidx2_v, rows_v, dv_v,
                    acc_sh, t_sh, sem, sem2, sem3, sem4):
    c = lax.axis_index("c")
    s = lax.axis_index("s")
    w = c * NS + s
    pltpu.sync_copy(z2_hbm.at[pl.ds(s * SL, SL)], acc_sh.at[pl.ds(s * SL, SL)])
    pltpu.sync_copy(z1_hbm.at[pl.ds(s * SL, SL)], t_sh.at[pl.ds(s * SL, SL)])
    plsc.subcore_barrier()

    def outer(i, carry):
        base = (w * RW + i * GS) * CHUNK
        pltpu.sync_copy(row1_hbm.at[pl.ds(base, GS * CHUNK)], ridx1_v)
        pltpu.sync_copy(col1_hbm.at[pl.ds(base, GS * CHUNK)], cidx1_v)
        cp1 = pltpu.async_copy(y_hbm.at[ridx1_v], rows_v, sem)
        cp2 = pltpu.async_copy(dinv_hbm.at[cidx1_v], dv_v, sem2)
        for j in range(GS):
            for k in range(CHUNK // 16):
                o = j * CHUNK + k * 16
                cidx2_v[j, pl.ds(k * 16, 16)] = cidx1_v[pl.ds(o, 16)]
                ridx2_v[j, pl.ds(k * 16, 16)] = ridx1_v[pl.ds(o, 16)]
        cp1.wait()
        cp2.wait()
        cps = []
        for j in range(GS):
            cps.append(pltpu.async_copy(
                rows_v.at[pl.ds(j * CHUNK, CHUNK)],
                acc_sh.at[cidx2_v.at[j]], sem3, add=True))
            cps.append(pltpu.async_copy(
                dv_v.at[pl.ds(j * CHUNK, CHUNK)],
                t_sh.at[ridx2_v.at[j]], sem4, add=True))
        for cp in cps:
            cp.wait()
        return carry

    lax.fori_loop(0, RW // GS, outer, 0)
    plsc.subcore_barrier()
    pltpu.sync_copy(acc_sh.at[pl.ds(s * SL, SL)], acc_out.at[c, pl.ds(s * SL, SL)])
    pltpu.sync_copy(t_sh.at[pl.ds(s * SL, SL)], t_out.at[c, pl.ds(s * SL, SL)])


def _scale_body(x_ref, w1_ref, degp_ref, y_ref, dinv_ref):
    deg = degp_ref[:, 0:1] + degp_ref[:, 1:2] + 1.0
    dinv = lax.rsqrt(deg)
    dinv_ref[...] = dinv
    xw = jnp.dot(x_ref[...], w1_ref[...], preferred_element_type=jnp.float32)
    y_ref[...] = xw * dinv


_scale_call = pl.pallas_call(
    _scale_body,
    out_shape=(
        jax.ShapeDtypeStruct((NPAD, H), jnp.float32),
        jax.ShapeDtypeStruct((NPAD, 1), jnp.float32),
    ),
)


def _final_body(accp_ref, tp_ref, y_ref, dinv_ref, b1_ref, w2_ref, b2_ref,
                lw_ref, lb_ref, out_ref):
    dinv = dinv_ref[...]
    acc = accp_ref[0] + accp_ref[1]
    h = jnp.maximum(dinv * (acc + y_ref[...]) + b1_ref[...], 0.0)
    t = tp_ref[:, 0:1] + tp_ref[:, 1:2]
    s = dinv * (dinv + t)
    ridx = lax.broadcasted_iota(jnp.int32, (NPAD, 1), 0)
    s = jnp.where(ridx < N, s, 0.0)
    z = jnp.sum(s * h, axis=0, keepdims=True)
    pooled = jnp.dot(z * (1.0 / N), w2_ref[...],
                     preferred_element_type=jnp.float32) + b2_ref[...]
    out_ref[...] = jnp.dot(pooled, lw_ref[...],
                           preferred_element_type=jnp.float32) + lb_ref[...]


_final_call = pl.pallas_call(
    _final_body,
    out_shape=jax.ShapeDtypeStruct((1, OUT), jnp.float32),
)


def kernel(x, edge_index, W1, b1, W2, b2, lin_W, lin_b):
    row = edge_index[0]
    col = edge_index[1]
    npad_extra = NPAD - N
    pad_idx = (N + jnp.arange(EP - E, dtype=jnp.int32) % npad_extra)
    rowp = jnp.concatenate([row, pad_idx])
    colp = jnp.concatenate([col, pad_idx])
    zeros1 = jnp.zeros((NPAD,), jnp.float32)
    zeros2 = jnp.zeros((NPAD, H), jnp.float32)

    degp = _deg_kernel(colp, zeros1)
    xp = jnp.pad(x, ((0, npad_extra), (0, 0)))
    y, dinv2 = _scale_call(xp, W1, degp.T)
    dinv1 = dinv2.reshape(NPAD)
    accp, tp = _scatter_kernel(rowp, colp, y, dinv1, zeros1, zeros2)
    out = _final_call(accp, tp.T, y, dinv2, b1.reshape(1, H), W2,
                      b2.reshape(1, H), lin_W, lin_b.reshape(1, OUT))
    return out.reshape(OUT)


# speedup vs baseline: 1.2301x; 1.1290x over previous
import functools

import jax
import jax.numpy as jnp
from jax import lax
from jax.experimental import pallas as pl
from jax.experimental.pallas import tpu as pltpu
from jax.experimental.pallas import tpu_sc as plsc

N = 10000
E = 320000
F_IN = 128
H = 16
OUT = 10

NC = 2
NS = 16
NW = NC * NS
NPAD = 10240
SL = NPAD // NS
CHUNK = 128
G = 8
EP = 327680
RW = EP // (NW * CHUNK)
RD = EP // (NS * CHUNK)

_mesh = plsc.VectorSubcoreMesh(core_axis_name="c", subcore_axis_name="s")


@functools.partial(
    pl.kernel,
    mesh=_mesh,
    out_type=(
        jax.ShapeDtypeStruct((NC, NPAD, H), jnp.float32),
        jax.ShapeDtypeStruct((NC, NPAD), jnp.float32),
        jax.ShapeDtypeStruct((NPAD,), jnp.float32),
    ),
    scratch_types=[
        pltpu.VMEM((G * CHUNK,), jnp.int32),
        pltpu.VMEM((G * CHUNK,), jnp.int32),
        pltpu.VMEM((G, CHUNK), jnp.int32),
        pltpu.VMEM((G, CHUNK), jnp.int32),
        pltpu.VMEM((G * CHUNK, H), jnp.float32),
        pltpu.VMEM((G * CHUNK,), jnp.float32),
        pltpu.VMEM((CHUNK,), jnp.float32),
        pltpu.VMEM((SL + 16,), jnp.float32),
        pltpu.VMEM((SL, H), jnp.float32),
        pltpu.VMEM_SHARED((NPAD,), jnp.float32),
        pltpu.VMEM_SHARED((NPAD,), jnp.float32),
        pltpu.VMEM_SHARED((NPAD, H), jnp.float32),
        pltpu.VMEM_SHARED((NPAD, H), jnp.float32),
        pltpu.VMEM_SHARED((NPAD,), jnp.float32),
        pltpu.SemaphoreType.DMA,
        pltpu.SemaphoreType.DMA,
        pltpu.SemaphoreType.DMA,
        pltpu.SemaphoreType.DMA,
    ],
    compiler_params=pltpu.CompilerParams(use_tc_tiling_on_sc=False,
                                         needs_layout_passes=False),
)
def _fused_kernel(row1_hbm, col1_hbm, xw_hbm, z1_hbm, z2_hbm,
                  acc_out, t_out, dinv_out,
                  ridx1_v, cidx1_v, ridx2_v, cidx2_v, rows_v, dv_v, ones_v,
                  dinvl_v, xwl_v,
                  deg_sh, dinv_sh, y_sh, acc_sh, t_sh,
                  sem, sem2, sem3, sem4):
    c = lax.axis_index("c")
    s = lax.axis_index("s")
    w = c * NS + s

    for k in range(CHUNK // 16):
        ones_v[pl.ds(k * 16, 16)] = jnp.ones((16,), jnp.float32)
    pltpu.sync_copy(z1_hbm.at[pl.ds(s * SL, SL)], deg_sh.at[pl.ds(s * SL, SL)])
    pltpu.sync_copy(z1_hbm.at[pl.ds(s * SL, SL)], t_sh.at[pl.ds(s * SL, SL)])
    pltpu.sync_copy(z2_hbm.at[pl.ds(s * SL, SL)], acc_sh.at[pl.ds(s * SL, SL)])
    plsc.subcore_barrier()

    def deg_outer(i, carry):
        base = (s * RD + i * G) * CHUNK
        pltpu.sync_copy(col1_hbm.at[pl.ds(base, G * CHUNK)], cidx1_v)
        for j in range(G):
            for k in range(CHUNK // 16):
                o = j * CHUNK + k * 16
                cidx2_v[j, pl.ds(k * 16, 16)] = cidx1_v[pl.ds(o, 16)]
        cps = []
        for j in range(G):
            cps.append(pltpu.async_copy(ones_v, deg_sh.at[cidx2_v.at[j]],
                                        sem, add=True))
        for cp in cps:
            cp.wait()
        return carry

    lax.fori_loop(0, RD // G, deg_outer, 0)
    plsc.subcore_barrier()

    pltpu.sync_copy(deg_sh.at[pl.ds(s * SL, SL)], dinvl_v.at[pl.ds(0, SL)])
    pltpu.sync_copy(xw_hbm.at[pl.ds(s * SL, SL)], xwl_v)

    def rsq(k, carry):
        d = dinvl_v[pl.ds(k * 16, 16)] + 1.0
        bits = plsc.bitcast(d, jnp.int32)
        bits = 0x5F3759DF - lax.shift_right_logical(bits, 1)
        r = plsc.bitcast(bits, jnp.float32)
        r = r * (1.5 - 0.5 * d * r * r)
        r = r * (1.5 - 0.5 * d * r * r)
        r = r * (1.5 - 0.5 * d * r * r)
        dinvl_v[pl.ds(k * 16, 16)] = r
        return carry

    lax.fori_loop(0, SL // 16, rsq, 0)

    def scale_row(r, carry):
        dvv = dinvl_v[pl.ds(r, 16)]
        xwl_v[r] = xwl_v[r] * dvv[0]
        return carry

    lax.fori_loop(0, SL, scale_row, 0)
    pltpu.sync_copy(dinvl_v.at[pl.ds(0, SL)], dinv_sh.at[pl.ds(s * SL, SL)])
    pltpu.sync_copy(xwl_v, y_sh.at[pl.ds(s * SL, SL)])

    @pl.when(c == 0)
    def _():
        pltpu.sync_copy(dinvl_v.at[pl.ds(0, SL)], dinv_out.at[pl.ds(s * SL, SL)])

    plsc.subcore_barrier()

    def outer(i, carry):
        base = (w * RW + i * G) * CHUNK
        pltpu.sync_copy(row1_hbm.at[pl.ds(base, G * CHUNK)], ridx1_v)
        pltpu.sync_copy(col1_hbm.at[pl.ds(base, G * CHUNK)], cidx1_v)
        cp1 = pltpu.async_copy(y_sh.at[ridx1_v], rows_v, sem)
        cp2 = pltpu.async_copy(dinv_sh.at[cidx1_v], dv_v, sem2)
        for j in range(G):
            for k in range(CHUNK // 16):
                o = j * CHUNK + k * 16
                cidx2_v[j, pl.ds(k * 16, 16)] = cidx1_v[pl.ds(o, 16)]
                ridx2_v[j, pl.ds(k * 16, 16)] = ridx1_v[pl.ds(o, 16)]
        cp1.wait()
        cp2.wait()
        cps = []
        for j in range(G):
            cps.append(pltpu.async_copy(
                rows_v.at[pl.ds(j * CHUNK, CHUNK)],
                acc_sh.at[cidx2_v.at[j]], sem3, add=True))
            cps.append(pltpu.async_copy(
                dv_v.at[pl.ds(j * CHUNK, CHUNK)],
                t_sh.at[ridx2_v.at[j]], sem4, add=True))
        for cp in cps:
            cp.wait()
        return carry

    lax.fori_loop(0, RW // G, outer, 0)
    plsc.subcore_barrier()

    pltpu.sync_copy(acc_sh.at[pl.ds(s * SL, SL)], acc_out.at[c, pl.ds(s * SL, SL)])
    pltpu.sync_copy(t_sh.at[pl.ds(s * SL, SL)], t_out.at[c, pl.ds(s * SL, SL)])


def _xw_body(x_ref, w1_ref, out_ref):
    out_ref[pl.ds(0, N), :] = jnp.dot(x_ref[...], w1_ref[...],
                                      preferred_element_type=jnp.float32)
    out_ref[pl.ds(N, NPAD - N), :] = jnp.zeros((NPAD - N, H), jnp.float32)


_xw_call = pl.pallas_call(
    _xw_body,
    out_shape=jax.ShapeDtypeStruct((NPAD, H), jnp.float32),
)


def _final_body(accp_ref, tp_ref, xw_ref, dinv_ref, b1_ref, w2_ref, b2_ref,
                lw_ref, lb_ref, out_ref):
    dinv = dinv_ref[...]
    acc = accp_ref[0] + accp_ref[1]
    y = dinv * xw_ref[...]
    h = jnp.maximum(dinv * (acc + y) + b1_ref[...], 0.0)
    t = tp_ref[:, 0:1] + tp_ref[:, 1:2]
    s = dinv * (dinv + t)
    ridx = lax.broadcasted_iota(jnp.int32, (NPAD, 1), 0)
    s = jnp.where(ridx < N, s, 0.0)
    z = jnp.sum(s * h, axis=0, keepdims=True)
    pooled = jnp.dot(z * (1.0 / N), w2_ref[...],
                     preferred_element_type=jnp.float32) + b2_ref[...]
    out_ref[...] = jnp.dot(pooled, lw_ref[...],
                           preferred_element_type=jnp.float32) + lb_ref[...]


_final_call = pl.pallas_call(
    _final_body,
    out_shape=jax.ShapeDtypeStruct((1, OUT), jnp.float32),
)


def kernel(x, edge_index, W1, b1, W2, b2, lin_W, lin_b):
    row = edge_index[0]
    col = edge_index[1]
    npad_extra = NPAD - N
    pad_idx = (N + jnp.arange(EP - E, dtype=jnp.int32) % npad_extra)
    rowp = jnp.concatenate([row, pad_idx])
    colp = jnp.concatenate([col, pad_idx])
    zeros1 = jnp.zeros((NPAD,), jnp.float32)
    zeros2 = jnp.zeros((NPAD, H), jnp.float32)

    xw = _xw_call(x, W1)
    accp, tp, dinv = _fused_kernel(rowp, colp, xw, zeros1, zeros2)
    out = _final_call(accp, tp.T, xw, dinv.reshape(NPAD, 1), b1.reshape(1, H),
                      W2, b2.reshape(1, H), lin_W, lin_b.reshape(1, OUT))
    return out.reshape(OUT)


# speedup vs baseline: 1.4053x; 1.1424x over previous
import functools

import jax
import jax.numpy as jnp
from jax import lax
from jax.experimental import pallas as pl
from jax.experimental.pallas import tpu as pltpu
from jax.experimental.pallas import tpu_sc as plsc

N = 10000
E = 320000
F_IN = 128
H = 16
OUT = 10

NC = 2
NS = 16
NW = NC * NS
NPAD = 10240
SL = NPAD // NS
CHUNK = 128
G = 8
EP = 327680
RW = EP // (NW * CHUNK)
RD = EP // (NS * CHUNK)

_mesh = plsc.VectorSubcoreMesh(core_axis_name="c", subcore_axis_name="s")


@functools.partial(
    pl.kernel,
    mesh=_mesh,
    out_type=(
        jax.ShapeDtypeStruct((NC, NPAD, H), jnp.float32),
        jax.ShapeDtypeStruct((NC, NPAD), jnp.float32),
        jax.ShapeDtypeStruct((NPAD,), jnp.float32),
    ),
    scratch_types=[
        pltpu.VMEM((RD * CHUNK,), jnp.int32),
        pltpu.VMEM((RW * CHUNK,), jnp.int32),
        pltpu.VMEM((RW * CHUNK,), jnp.int32),
        pltpu.VMEM((G, CHUNK), jnp.int32),
        pltpu.VMEM((G, CHUNK), jnp.int32),
        pltpu.VMEM((G * CHUNK, H), jnp.float32),
        pltpu.VMEM((G * CHUNK,), jnp.float32),
        pltpu.VMEM((CHUNK,), jnp.float32),
        pltpu.VMEM((SL + 16,), jnp.float32),
        pltpu.VMEM((SL, H), jnp.float32),
        pltpu.VMEM_SHARED((NPAD,), jnp.float32),
        pltpu.VMEM_SHARED((NPAD,), jnp.float32),
        pltpu.VMEM_SHARED((NPAD, H), jnp.float32),
        pltpu.VMEM_SHARED((NPAD, H), jnp.float32),
        pltpu.VMEM_SHARED((NPAD,), jnp.float32),
        pltpu.SemaphoreType.DMA,
        pltpu.SemaphoreType.DMA,
        pltpu.SemaphoreType.DMA,
        pltpu.SemaphoreType.DMA,
        pltpu.SemaphoreType.DMA,
        pltpu.SemaphoreType.DMA,
    ],
    compiler_params=pltpu.CompilerParams(use_tc_tiling_on_sc=False,
                                         needs_layout_passes=False),
)
def _fused_kernel(row1_hbm, col1_hbm, xw_hbm, z1_hbm, z2_hbm,
                  acc_out, t_out, dinv_out,
                  dcol_v, ridx1_v, cidx1_v, ridx2_v, cidx2_v, rows_v, dv_v,
                  ones_v, dinvl_v, xwl_v,
                  deg_sh, dinv_sh, y_sh, acc_sh, t_sh,
                  sem, sem2, sem3, sem4, semp, semq):
    c = lax.axis_index("c")
    s = lax.axis_index("s")
    w = c * NS + s

    cpd = pltpu.async_copy(
        col1_hbm.at[pl.ds(s * RD * CHUNK, RD * CHUNK)], dcol_v, semp)
    cpr = pltpu.async_copy(
        row1_hbm.at[pl.ds(w * RW * CHUNK, RW * CHUNK)], ridx1_v, semq)
    cpc = pltpu.async_copy(
        col1_hbm.at[pl.ds(w * RW * CHUNK, RW * CHUNK)], cidx1_v, semq)
    cpx = pltpu.async_copy(xw_hbm.at[pl.ds(s * SL, SL)], xwl_v, semq)

    for k in range(CHUNK // 16):
        ones_v[pl.ds(k * 16, 16)] = jnp.ones((16,), jnp.float32)
    pltpu.sync_copy(z1_hbm.at[pl.ds(s * SL, SL)], deg_sh.at[pl.ds(s * SL, SL)])
    pltpu.sync_copy(z1_hbm.at[pl.ds(s * SL, SL)], t_sh.at[pl.ds(s * SL, SL)])
    pltpu.sync_copy(z2_hbm.at[pl.ds(s * SL, SL)], acc_sh.at[pl.ds(s * SL, SL)])
    plsc.subcore_barrier()

    cpd.wait()

    def deg_outer(i, carry):
        for j in range(G):
            for k in range(CHUNK // 16):
                o = (i * G + j) * CHUNK + k * 16
                cidx2_v[j, pl.ds(k * 16, 16)] = dcol_v[pl.ds(o, 16)]
        cps = []
        for j in range(G):
            cps.append(pltpu.async_copy(ones_v, deg_sh.at[cidx2_v.at[j]],
                                        sem, add=True))
        for cp in cps:
            cp.wait()
        return carry

    lax.fori_loop(0, RD // G, deg_outer, 0)
    plsc.subcore_barrier()

    pltpu.sync_copy(deg_sh.at[pl.ds(s * SL, SL)], dinvl_v.at[pl.ds(0, SL)])
    cpx.wait()
    cpr.wait()
    cpc.wait()

    def rsq(k, carry):
        d = dinvl_v[pl.ds(k * 16, 16)] + 1.0
        bits = plsc.bitcast(d, jnp.int32)
        bits = 0x5F3759DF - lax.shift_right_logical(bits, 1)
        r = plsc.bitcast(bits, jnp.float32)
        r = r * (1.5 - 0.5 * d * r * r)
        r = r * (1.5 - 0.5 * d * r * r)
        r = r * (1.5 - 0.5 * d * r * r)
        dinvl_v[pl.ds(k * 16, 16)] = r
        return carry

    lax.fori_loop(0, SL // 16, rsq, 0)

    def scale_row(r, carry):
        dvv = dinvl_v[pl.ds(r, 16)]
        xwl_v[r] = xwl_v[r] * dvv[0]
        return carry

    lax.fori_loop(0, SL, scale_row, 0)
    pltpu.sync_copy(dinvl_v.at[pl.ds(0, SL)], dinv_sh.at[pl.ds(s * SL, SL)])
    pltpu.sync_copy(xwl_v, y_sh.at[pl.ds(s * SL, SL)])

    @pl.when(c == 0)
    def _():
        pltpu.sync_copy(dinvl_v.at[pl.ds(0, SL)], dinv_out.at[pl.ds(s * SL, SL)])

    plsc.subcore_barrier()

    def outer(i, carry):
        base = i * G * CHUNK
        cp1 = pltpu.async_copy(
            y_sh.at[ridx1_v.at[pl.ds(base, G * CHUNK)]], rows_v, sem)
        cp2 = pltpu.async_copy(
            dinv_sh.at[cidx1_v.at[pl.ds(base, G * CHUNK)]], dv_v, sem2)
        for j in range(G):
            for k in range(CHUNK // 16):
                o = base + j * CHUNK + k * 16
                cidx2_v[j, pl.ds(k * 16, 16)] = cidx1_v[pl.ds(o, 16)]
                ridx2_v[j, pl.ds(k * 16, 16)] = ridx1_v[pl.ds(o, 16)]
        cp1.wait()
        cp2.wait()
        cps = []
        for j in range(G):
            cps.append(pltpu.async_copy(
                rows_v.at[pl.ds(j * CHUNK, CHUNK)],
                acc_sh.at[cidx2_v.at[j]], sem3, add=True))
            cps.append(pltpu.async_copy(
                dv_v.at[pl.ds(j * CHUNK, CHUNK)],
                t_sh.at[ridx2_v.at[j]], sem4, add=True))
        for cp in cps:
            cp.wait()
        return carry

    lax.fori_loop(0, RW // G, outer, 0)
    plsc.subcore_barrier()

    pltpu.sync_copy(acc_sh.at[pl.ds(s * SL, SL)], acc_out.at[c, pl.ds(s * SL, SL)])
    pltpu.sync_copy(t_sh.at[pl.ds(s * SL, SL)], t_out.at[c, pl.ds(s * SL, SL)])


def _xw_body(x_ref, w1_ref, out_ref):
    out_ref[pl.ds(0, N), :] = jnp.dot(x_ref[...], w1_ref[...],
                                      preferred_element_type=jnp.float32)
    out_ref[pl.ds(N, NPAD - N), :] = jnp.zeros((NPAD - N, H), jnp.float32)


_xw_call = pl.pallas_call(
    _xw_body,
    out_shape=jax.ShapeDtypeStruct((NPAD, H), jnp.float32),
)


def _final_body(accp_ref, tp_ref, xw_ref, dinv_ref, b1_ref, w2_ref, b2_ref,
                lw_ref, lb_ref, out_ref):
    dinv = dinv_ref[...]
    acc = accp_ref[0] + accp_ref[1]
    y = dinv * xw_ref[...]
    h = jnp.maximum(dinv * (acc + y) + b1_ref[...], 0.0)
    t = tp_ref[:, 0:1] + tp_ref[:, 1:2]
    s = dinv * (dinv + t)
    ridx = lax.broadcasted_iota(jnp.int32, (NPAD, 1), 0)
    s = jnp.where(ridx < N, s, 0.0)
    z = jnp.sum(s * h, axis=0, keepdims=True)
    pooled = jnp.dot(z * (1.0 / N), w2_ref[...],
                     preferred_element_type=jnp.float32) + b2_ref[...]
    out_ref[...] = jnp.dot(pooled, lw_ref[...],
                           preferred_element_type=jnp.float32) + lb_ref[...]


_final_call = pl.pallas_call(
    _final_body,
    out_shape=jax.ShapeDtypeStruct((1, OUT), jnp.float32),
)


def kernel(x, edge_index, W1, b1, W2, b2, lin_W, lin_b):
    row = edge_index[0]
    col = edge_index[1]
    npad_extra = NPAD - N
    pad_idx = (N + jnp.arange(EP - E, dtype=jnp.int32) % npad_extra)
    rowp = jnp.concatenate([row, pad_idx])
    colp = jnp.concatenate([col, pad_idx])
    zeros1 = jnp.zeros((NPAD,), jnp.float32)
    zeros2 = jnp.zeros((NPAD, H), jnp.float32)

    xw = _xw_call(x, W1)
    accp, tp, dinv = _fused_kernel(rowp, colp, xw, zeros1, zeros2)
    out = _final_call(accp, tp.T, xw, dinv.reshape(NPAD, 1), b1.reshape(1, H),
                      W2, b2.reshape(1, H), lin_W, lin_b.reshape(1, OUT))
    return out.reshape(OUT)


# speedup vs baseline: 1.4293x; 1.0171x over previous
import functools

import jax
import jax.numpy as jnp
from jax import lax
from jax.experimental import pallas as pl
from jax.experimental.pallas import tpu as pltpu
from jax.experimental.pallas import tpu_sc as plsc

N = 10000
E = 320000
F_IN = 128
H = 16
OUT = 10

NC = 2
NS = 16
NW = NC * NS
NPAD = 10240
SL = NPAD // NS
CHUNK = 128
G = 8
GM = 5
EP = 327680
RW = EP // (NW * CHUNK)
RD = EP // (NS * CHUNK)

_mesh = plsc.VectorSubcoreMesh(core_axis_name="c", subcore_axis_name="s")


@functools.partial(
    pl.kernel,
    mesh=_mesh,
    out_type=(
        jax.ShapeDtypeStruct((NC, NPAD, H), jnp.float32),
        jax.ShapeDtypeStruct((NC, NPAD), jnp.float32),
        jax.ShapeDtypeStruct((NPAD,), jnp.float32),
    ),
    scratch_types=[
        pltpu.VMEM((RD * CHUNK,), jnp.int32),
        pltpu.VMEM((RW * CHUNK,), jnp.int32),
        pltpu.VMEM((RW * CHUNK,), jnp.int32),
        pltpu.VMEM((G, CHUNK), jnp.int32),
        pltpu.VMEM((G, CHUNK), jnp.int32),
        pltpu.VMEM((GM, CHUNK), jnp.int32),
        pltpu.VMEM((GM, CHUNK), jnp.int32),
        pltpu.VMEM((GM, CHUNK), jnp.int32),
        pltpu.VMEM((GM, CHUNK), jnp.int32),
        pltpu.VMEM((GM * CHUNK, H), jnp.float32),
        pltpu.VMEM((GM * CHUNK, H), jnp.float32),
        pltpu.VMEM((GM * CHUNK,), jnp.float32),
        pltpu.VMEM((GM * CHUNK,), jnp.float32),
        pltpu.VMEM((CHUNK,), jnp.float32),
        pltpu.VMEM((SL + 16,), jnp.float32),
        pltpu.VMEM((SL, H), jnp.float32),
        pltpu.VMEM_SHARED((NPAD,), jnp.float32),
        pltpu.VMEM_SHARED((NPAD,), jnp.float32),
        pltpu.VMEM_SHARED((NPAD, H), jnp.float32),
        pltpu.VMEM_SHARED((NPAD, H), jnp.float32),
        pltpu.VMEM_SHARED((NPAD,), jnp.float32),
        pltpu.SemaphoreType.DMA,
        pltpu.SemaphoreType.DMA,
        pltpu.SemaphoreType.DMA,
        pltpu.SemaphoreType.DMA,
        pltpu.SemaphoreType.DMA,
        pltpu.SemaphoreType.DMA,
        pltpu.SemaphoreType.DMA,
    ],
    compiler_params=pltpu.CompilerParams(use_tc_tiling_on_sc=False,
                                         needs_layout_passes=False),
)
def _fused_kernel(row1_hbm, col1_hbm, xw_hbm, z1_hbm, z2_hbm,
                  acc_out, t_out, dinv_out,
                  dcol_v, ridx1_v, cidx1_v, d2a_v, d2b_v,
                  r2a_v, c2a_v, r2b_v, c2b_v, rowsa_v, rowsb_v, dva_v, dvb_v,
                  ones_v, dinvl_v, xwl_v,
                  deg_sh, dinv_sh, y_sh, acc_sh, t_sh,
                  semga, semgb, semsa, semsb, semdeg, semp, semq):
    c = lax.axis_index("c")
    s = lax.axis_index("s")
    w = c * NS + s

    cpd = pltpu.async_copy(
        col1_hbm.at[pl.ds(s * RD * CHUNK, RD * CHUNK)], dcol_v, semp)
    cpr = pltpu.async_copy(
        row1_hbm.at[pl.ds(w * RW * CHUNK, RW * CHUNK)], ridx1_v, semq)
    cpc = pltpu.async_copy(
        col1_hbm.at[pl.ds(w * RW * CHUNK, RW * CHUNK)], cidx1_v, semq)
    cpx = pltpu.async_copy(xw_hbm.at[pl.ds(s * SL, SL)], xwl_v, semq)

    for k in range(CHUNK // 16):
        ones_v[pl.ds(k * 16, 16)] = jnp.ones((16,), jnp.float32)
    pltpu.sync_copy(z1_hbm.at[pl.ds(s * SL, SL)], deg_sh.at[pl.ds(s * SL, SL)])
    pltpu.sync_copy(z1_hbm.at[pl.ds(s * SL, SL)], t_sh.at[pl.ds(s * SL, SL)])
    pltpu.sync_copy(z2_hbm.at[pl.ds(s * SL, SL)], acc_sh.at[pl.ds(s * SL, SL)])
    plsc.subcore_barrier()

    cpd.wait()

    def deg_outer(i, carry):
        cps = []
        for buf, g in ((d2a_v, 0), (d2b_v, 1)):
            for j in range(G):
                for k in range(CHUNK // 16):
                    o = ((i * 2 + g) * G + j) * CHUNK + k * 16
                    buf[j, pl.ds(k * 16, 16)] = dcol_v[pl.ds(o, 16)]
            for j in range(G):
                cps.append(pltpu.async_copy(ones_v, deg_sh.at[buf.at[j]],
                                            semdeg, add=True))
        for cp in cps:
            cp.wait()
        return carry

    lax.fori_loop(0, RD // (2 * G), deg_outer, 0)
    plsc.subcore_barrier()

    pltpu.sync_copy(deg_sh.at[pl.ds(s * SL, SL)], dinvl_v.at[pl.ds(0, SL)])
    cpx.wait()
    cpr.wait()
    cpc.wait()

    def rsq(k, carry):
        d = dinvl_v[pl.ds(k * 16, 16)] + 1.0
        bits = plsc.bitcast(d, jnp.int32)
        bits = 0x5F3759DF - lax.shift_right_logical(bits, 1)
        r = plsc.bitcast(bits, jnp.float32)
        r = r * (1.5 - 0.5 * d * r * r)
        r = r * (1.5 - 0.5 * d * r * r)
        r = r * (1.5 - 0.5 * d * r * r)
        dinvl_v[pl.ds(k * 16, 16)] = r
        return carry

    lax.fori_loop(0, SL // 16, rsq, 0)

    def scale_row(r, carry):
        dvv = dinvl_v[pl.ds(r, 16)]
        xwl_v[r] = xwl_v[r] * dvv[0]
        return carry

    lax.fori_loop(0, SL, scale_row, 0)
    pltpu.sync_copy(dinvl_v.at[pl.ds(0, SL)], dinv_sh.at[pl.ds(s * SL, SL)])
    pltpu.sync_copy(xwl_v, y_sh.at[pl.ds(s * SL, SL)])

    @pl.when(c == 0)
    def _():
        pltpu.sync_copy(dinvl_v.at[pl.ds(0, SL)], dinv_out.at[pl.ds(s * SL, SL)])

    plsc.subcore_barrier()

    def _fire_gathers(base, rows_buf, dv_buf, semg):
        cpg = pltpu.async_copy(
            y_sh.at[ridx1_v.at[pl.ds(base, GM * CHUNK)]], rows_buf, semg)
        cpd2 = pltpu.async_copy(
            dinv_sh.at[cidx1_v.at[pl.ds(base, GM * CHUNK)]], dv_buf, semg)
        return cpg, cpd2

    def _copy_idx(base, r2_buf, c2_buf):
        for j in range(GM):
            for k in range(CHUNK // 16):
                o = base + j * CHUNK + k * 16
                c2_buf[j, pl.ds(k * 16, 16)] = cidx1_v[pl.ds(o, 16)]
                r2_buf[j, pl.ds(k * 16, 16)] = ridx1_v[pl.ds(o, 16)]

    def _fire_scatters(rows_buf, dv_buf, r2_buf, c2_buf, sems):
        cps = []
        for j in range(GM):
            cps.append(pltpu.async_copy(
                rows_buf.at[pl.ds(j * CHUNK, CHUNK)],
                acc_sh.at[c2_buf.at[j]], sems, add=True))
            cps.append(pltpu.async_copy(
                dv_buf.at[pl.ds(j * CHUNK, CHUNK)],
                t_sh.at[r2_buf.at[j]], sems, add=True))
        return cps

    def outer(i, carry):
        ba = (i * 2) * GM * CHUNK
        bb = ba + GM * CHUNK
        cpga, cpgda = _fire_gathers(ba, rowsa_v, dva_v, semga)
        _copy_idx(ba, r2a_v, c2a_v)
        cpga.wait()
        cpgda.wait()
        cpsa = _fire_scatters(rowsa_v, dva_v, r2a_v, c2a_v, semsa)
        cpgb, cpgdb = _fire_gathers(bb, rowsb_v, dvb_v, semgb)
        _copy_idx(bb, r2b_v, c2b_v)
        cpgb.wait()
        cpgdb.wait()
        cpsb = _fire_scatters(rowsb_v, dvb_v, r2b_v, c2b_v, semsb)
        for cp in cpsa:
            cp.wait()
        for cp in cpsb:
            cp.wait()
        return carry

    lax.fori_loop(0, RW // (2 * GM), outer, 0)
    plsc.subcore_barrier()

    pltpu.sync_copy(acc_sh.at[pl.ds(s * SL, SL)], acc_out.at[c, pl.ds(s * SL, SL)])
    pltpu.sync_copy(t_sh.at[pl.ds(s * SL, SL)], t_out.at[c, pl.ds(s * SL, SL)])


def _xw_body(x_ref, w1_ref, out_ref):
    out_ref[pl.ds(0, N), :] = jnp.dot(x_ref[...], w1_ref[...],
                                      preferred_element_type=jnp.float32)
    out_ref[pl.ds(N, NPAD - N), :] = jnp.zeros((NPAD - N, H), jnp.float32)


_xw_call = pl.pallas_call(
    _xw_body,
    out_shape=jax.ShapeDtypeStruct((NPAD, H), jnp.float32),
)


def _final_body(accp_ref, tp_ref, xw_ref, dinv_ref, b1_ref, w2_ref, b2_ref,
                lw_ref, lb_ref, out_ref):
    dinv = dinv_ref[...]
    acc = accp_ref[0] + accp_ref[1]
    y = dinv * xw_ref[...]
    h = jnp.maximum(dinv * (acc + y) + b1_ref[...], 0.0)
    t = tp_ref[:, 0:1] + tp_ref[:, 1:2]
    s = dinv * (dinv + t)
    ridx = lax.broadcasted_iota(jnp.int32, (NPAD, 1), 0)
    s = jnp.where(ridx < N, s, 0.0)
    z = jnp.sum(s * h, axis=0, keepdims=True)
    pooled = jnp.dot(z * (1.0 / N), w2_ref[...],
                     preferred_element_type=jnp.float32) + b2_ref[...]
    out_ref[...] = jnp.dot(pooled, lw_ref[...],
                           preferred_element_type=jnp.float32) + lb_ref[...]


_final_call = pl.pallas_call(
    _final_body,
    out_shape=jax.ShapeDtypeStruct((1, OUT), jnp.float32),
)


def kernel(x, edge_index, W1, b1, W2, b2, lin_W, lin_b):
    row = edge_index[0]
    col = edge_index[1]
    npad_extra = NPAD - N
    pad_idx = (N + jnp.arange(EP - E, dtype=jnp.int32) % npad_extra)
    rowp = jnp.concatenate([row, pad_idx])
    colp = jnp.concatenate([col, pad_idx])
    zeros1 = jnp.zeros((NPAD,), jnp.float32)
    zeros2 = jnp.zeros((NPAD, H), jnp.float32)

    xw = _xw_call(x, W1)
    accp, tp, dinv = _fused_kernel(rowp, colp, xw, zeros1, zeros2)
    out = _final_call(accp, tp.T, xw, dinv.reshape(NPAD, 1), b1.reshape(1, H),
                      W2, b2.reshape(1, H), lin_W, lin_b.reshape(1, OUT))
    return out.reshape(OUT)


# speedup vs baseline: 1.9575x; 1.3695x over previous
import functools

import jax
import jax.numpy as jnp
from jax import lax
from jax.experimental import pallas as pl
from jax.experimental.pallas import tpu as pltpu
from jax.experimental.pallas import tpu_sc as plsc

N = 10000
E = 320000
F_IN = 128
H = 16
OUT = 10

NC = 2
NS = 16
NW = NC * NS
NPAD = 10240
SL = NPAD // NS
CHUNK = 128
G = 8
GM = 5
EP = 327680
RW = EP // (NW * CHUNK)
RD = EP // (NS * CHUNK)

_mesh = plsc.VectorSubcoreMesh(core_axis_name="c", subcore_axis_name="s")


@functools.partial(
    pl.kernel,
    mesh=_mesh,
    out_type=(
        jax.ShapeDtypeStruct((NC, NPAD, H), jnp.float32),
        jax.ShapeDtypeStruct((NC * NPAD,), jnp.float32),
        jax.ShapeDtypeStruct((NPAD,), jnp.float32),
        jax.ShapeDtypeStruct((NPAD, H), jnp.float32),
    ),
    scratch_types=[
        pltpu.VMEM((RD * CHUNK,), jnp.int32),
        pltpu.VMEM((RW * CHUNK,), jnp.int32),
        pltpu.VMEM((RW * CHUNK,), jnp.int32),
        pltpu.VMEM((G, CHUNK), jnp.int32),
        pltpu.VMEM((G, CHUNK), jnp.int32),
        pltpu.VMEM((GM, CHUNK), jnp.int32),
        pltpu.VMEM((GM, CHUNK), jnp.int32),
        pltpu.VMEM((GM, CHUNK), jnp.int32),
        pltpu.VMEM((GM, CHUNK), jnp.int32),
        pltpu.VMEM((GM * CHUNK, H), jnp.float32),
        pltpu.VMEM((GM * CHUNK, H), jnp.float32),
        pltpu.VMEM((GM * CHUNK,), jnp.float32),
        pltpu.VMEM((GM * CHUNK,), jnp.float32),
        pltpu.VMEM((CHUNK,), jnp.float32),
        pltpu.VMEM((SL + 16,), jnp.float32),
        pltpu.VMEM((SL, H), jnp.float32),
        pltpu.VMEM_SHARED((NPAD,), jnp.float32),
        pltpu.VMEM_SHARED((NPAD,), jnp.float32),
        pltpu.VMEM_SHARED((NPAD, H), jnp.float32),
        pltpu.VMEM_SHARED((NPAD, H), jnp.float32),
        pltpu.VMEM_SHARED((NPAD,), jnp.float32),
        pltpu.SemaphoreType.DMA,
        pltpu.SemaphoreType.DMA,
        pltpu.SemaphoreType.DMA,
        pltpu.SemaphoreType.DMA,
        pltpu.SemaphoreType.DMA,
        pltpu.SemaphoreType.DMA,
        pltpu.SemaphoreType.DMA,
    ],
    compiler_params=pltpu.CompilerParams(use_tc_tiling_on_sc=False,
                                         needs_layout_passes=False),
)
def _fused_kernel(row1_hbm, col1_hbm, xw_hbm, z1_hbm, z2_hbm,
                  acc_out, t_out, dinv_out, y_out,
                  dcol_v, ridx1_v, cidx1_v, d2a_v, d2b_v,
                  r2a_v, c2a_v, r2b_v, c2b_v, rowsa_v, rowsb_v, dva_v, dvb_v,
                  ones_v, dinvl_v, xwl_v,
                  deg_sh, dinv_sh, y_sh, acc_sh, t_sh,
                  semga, semgb, semsa, semsb, semdeg, semp, semq):
    c = lax.axis_index("c")
    s = lax.axis_index("s")
    w = c * NS + s

    cpd = pltpu.async_copy(
        col1_hbm.at[pl.ds(s * RD * CHUNK, RD * CHUNK)], dcol_v, semp)
    cpr = pltpu.async_copy(
        row1_hbm.at[pl.ds(w * RW * CHUNK, RW * CHUNK)], ridx1_v, semq)
    cpc = pltpu.async_copy(
        col1_hbm.at[pl.ds(w * RW * CHUNK, RW * CHUNK)], cidx1_v, semq)
    cpx = pltpu.async_copy(xw_hbm.at[pl.ds(s * SL, SL)], xwl_v, semq)

    for k in range(CHUNK // 16):
        ones_v[pl.ds(k * 16, 16)] = jnp.ones((16,), jnp.float32)
    pltpu.sync_copy(z1_hbm.at[pl.ds(s * SL, SL)], deg_sh.at[pl.ds(s * SL, SL)])
    pltpu.sync_copy(z1_hbm.at[pl.ds(s * SL, SL)], t_sh.at[pl.ds(s * SL, SL)])
    pltpu.sync_copy(z2_hbm.at[pl.ds(s * SL, SL)], acc_sh.at[pl.ds(s * SL, SL)])
    plsc.subcore_barrier()

    cpd.wait()

    def deg_outer(i, carry):
        cps = []
        for buf, g in ((d2a_v, 0), (d2b_v, 1)):
            for j in range(G):
                for k in range(CHUNK // 16):
                    o = ((i * 2 + g) * G + j) * CHUNK + k * 16
                    buf[j, pl.ds(k * 16, 16)] = dcol_v[pl.ds(o, 16)]
            for j in range(G):
                cps.append(pltpu.async_copy(ones_v, deg_sh.at[buf.at[j]],
                                            semdeg, add=True))
        for cp in cps:
            cp.wait()
        return carry

    lax.fori_loop(0, RD // (2 * G), deg_outer, 0)
    plsc.subcore_barrier()

    pltpu.sync_copy(deg_sh.at[pl.ds(s * SL, SL)], dinvl_v.at[pl.ds(0, SL)])
    cpx.wait()
    cpr.wait()
    cpc.wait()

    def rsq(k, carry):
        d = dinvl_v[pl.ds(k * 16, 16)] + 1.0
        bits = plsc.bitcast(d, jnp.int32)
        bits = 0x5F3759DF - lax.shift_right_logical(bits, 1)
        r = plsc.bitcast(bits, jnp.float32)
        r = r * (1.5 - 0.5 * d * r * r)
        r = r * (1.5 - 0.5 * d * r * r)
        r = r * (1.5 - 0.5 * d * r * r)
        dinvl_v[pl.ds(k * 16, 16)] = r
        return carry

    lax.fori_loop(0, SL // 16, rsq, 0)

    def scale_row(r, carry):
        dvv = dinvl_v[pl.ds(r, 16)]
        xwl_v[r] = xwl_v[r] * dvv[0]
        return carry

    lax.fori_loop(0, SL, scale_row, 0)
    pltpu.sync_copy(dinvl_v.at[pl.ds(0, SL)], dinv_sh.at[pl.ds(s * SL, SL)])
    pltpu.sync_copy(xwl_v, y_sh.at[pl.ds(s * SL, SL)])

    @pl.when(c == 0)
    def _():
        pltpu.sync_copy(dinvl_v.at[pl.ds(0, SL)], dinv_out.at[pl.ds(s * SL, SL)])
        pltpu.sync_copy(xwl_v, y_out.at[pl.ds(s * SL, SL)])

    plsc.subcore_barrier()

    def _fire_gathers(base, rows_buf, dv_buf, semg):
        cpg = pltpu.async_copy(
            y_sh.at[ridx1_v.at[pl.ds(base, GM * CHUNK)]], rows_buf, semg)
        cpd2 = pltpu.async_copy(
            dinv_sh.at[cidx1_v.at[pl.ds(base, GM * CHUNK)]], dv_buf, semg)
        return cpg, cpd2

    def _copy_idx(base, r2_buf, c2_buf):
        for j in range(GM):
            for k in range(CHUNK // 16):
                o = base + j * CHUNK + k * 16
                c2_buf[j, pl.ds(k * 16, 16)] = cidx1_v[pl.ds(o, 16)]
                r2_buf[j, pl.ds(k * 16, 16)] = ridx1_v[pl.ds(o, 16)]

    def _fire_scatters(rows_buf, dv_buf, r2_buf, c2_buf, sems):
        cps = []
        for j in range(GM):
            cps.append(pltpu.async_copy(
                rows_buf.at[pl.ds(j * CHUNK, CHUNK)],
                acc_sh.at[c2_buf.at[j]], sems, add=True))
            cps.append(pltpu.async_copy(
                dv_buf.at[pl.ds(j * CHUNK, CHUNK)],
                t_sh.at[r2_buf.at[j]], sems, add=True))
        return cps

    def outer(i, carry):
        ba = (i * 2) * GM * CHUNK
        bb = ba + GM * CHUNK
        cpga, cpgda = _fire_gathers(ba, rowsa_v, dva_v, semga)
        _copy_idx(ba, r2a_v, c2a_v)
        cpga.wait()
        cpgda.wait()
        cpsa = _fire_scatters(rowsa_v, dva_v, r2a_v, c2a_v, semsa)
        cpgb, cpgdb = _fire_gathers(bb, rowsb_v, dvb_v, semgb)
        _copy_idx(bb, r2b_v, c2b_v)
        cpgb.wait()
        cpgdb.wait()
        cpsb = _fire_scatters(rowsb_v, dvb_v, r2b_v, c2b_v, semsb)
        for cp in cpsa:
            cp.wait()
        for cp in cpsb:
            cp.wait()
        return carry

    lax.fori_loop(0, RW // (2 * GM), outer, 0)
    plsc.subcore_barrier()

    pltpu.sync_copy(acc_sh.at[pl.ds(s * SL, SL)], acc_out.at[c, pl.ds(s * SL, SL)])
    pltpu.sync_copy(t_sh.at[pl.ds(s * SL, SL)],
                    t_out.at[pl.ds(c * NPAD + s * SL, SL)])


def _prep_body(ei_ref, pad_ref, x_ref, w1_ref, rowp_ref, colp_ref, xw_ref):
    ei = ei_ref[...]
    pad = pad_ref[...]
    rowp_ref[...] = jnp.concatenate([ei[0], pad])
    colp_ref[...] = jnp.concatenate([ei[1], pad])
    xw_ref[pl.ds(0, N), :] = jnp.dot(x_ref[...], w1_ref[...],
                                     preferred_element_type=jnp.float32)
    xw_ref[pl.ds(N, NPAD - N), :] = jnp.zeros((NPAD - N, H), jnp.float32)


_prep_call = pl.pallas_call(
    _prep_body,
    out_shape=(
        jax.ShapeDtypeStruct((EP,), jnp.int32),
        jax.ShapeDtypeStruct((EP,), jnp.int32),
        jax.ShapeDtypeStruct((NPAD, H), jnp.float32),
    ),
)


PK = NPAD // 8


def _final_body(accp_ref, y_ref, dinv_ref, s_ref, b1_ref, w2_ref, b2_ref,
                lw_ref, lb_ref, out_ref):
    acc = accp_ref[0] + accp_ref[1]
    h = jnp.maximum(dinv_ref[...] * (acc + y_ref[...]) + b1_ref[...], 0.0)
    q = s_ref[...] * h
    z128 = jnp.sum(q, axis=0, keepdims=True)
    z = z128[:, 0:H]
    for g in range(1, 8):
        z = z + z128[:, g * H:(g + 1) * H]
    pooled = jnp.dot(z * (1.0 / N), w2_ref[...],
                     preferred_element_type=jnp.float32) + b2_ref[...]
    out_ref[...] = jnp.dot(pooled, lw_ref[...],
                           preferred_element_type=jnp.float32) + lb_ref[...]


_final_call = pl.pallas_call(
    _final_body,
    out_shape=jax.ShapeDtypeStruct((1, OUT), jnp.float32),
)


def kernel(x, edge_index, W1, b1, W2, b2, lin_W, lin_b):
    npad_extra = NPAD - N
    pad_idx = (N + jnp.arange(EP - E, dtype=jnp.int32) % npad_extra)
    zeros1 = jnp.zeros((NPAD,), jnp.float32)
    zeros2 = jnp.zeros((NPAD, H), jnp.float32)

    rowp, colp, xw = _prep_call(edge_index, pad_idx, x, W1)
    accp, t1d, dinv, y = _fused_kernel(rowp, colp, xw, zeros1, zeros2)

    tp = t1d.reshape(NC, NPAD)
    t = tp[0] + tp[1]
    s = dinv * (dinv + t)
    s = jnp.where(jnp.arange(NPAD) < N, s, 0.0)
    s_exp = jnp.repeat(s, H).reshape(PK, 128)
    dinv_exp = jnp.repeat(dinv, H).reshape(PK, 128)
    accp2 = accp.reshape(NC, PK, 128)
    y2 = y.reshape(PK, 128)
    b1t = jnp.tile(b1, 8).reshape(1, 128)

    out = _final_call(accp2, y2, dinv_exp, s_exp, b1t,
                      W2, b2.reshape(1, H), lin_W, lin_b.reshape(1, OUT))
    return out.reshape(OUT)


# speedup vs baseline: 2.0197x; 1.0318x over previous
import functools

import jax
import jax.numpy as jnp
from jax import lax
from jax.experimental import pallas as pl
from jax.experimental.pallas import tpu as pltpu
from jax.experimental.pallas import tpu_sc as plsc

N = 10000
E = 320000
F_IN = 128
H = 16
OUT = 10

NC = 2
NS = 16
NW = NC * NS
NPAD = 10240
SL = NPAD // NS
CHUNK = 128
G = 8
GM = 5
EP = 327680
RW = EP // (NW * CHUNK)
RD = EP // (NS * CHUNK)

_mesh = plsc.VectorSubcoreMesh(core_axis_name="c", subcore_axis_name="s")


@functools.partial(
    pl.kernel,
    mesh=_mesh,
    out_type=(
        jax.ShapeDtypeStruct((NC, NPAD, H), jnp.float32),
        jax.ShapeDtypeStruct((NC * NPAD,), jnp.float32),
        jax.ShapeDtypeStruct((NPAD,), jnp.float32),
        jax.ShapeDtypeStruct((NPAD, H), jnp.float32),
    ),
    scratch_types=[
        pltpu.VMEM((RD * CHUNK,), jnp.int32),
        pltpu.VMEM((RW * CHUNK,), jnp.int32),
        pltpu.VMEM((RW * CHUNK,), jnp.int32),
        pltpu.VMEM((G, CHUNK), jnp.int32),
        pltpu.VMEM((G, CHUNK), jnp.int32),
        pltpu.VMEM((GM, CHUNK), jnp.int32),
        pltpu.VMEM((GM, CHUNK), jnp.int32),
        pltpu.VMEM((GM, CHUNK), jnp.int32),
        pltpu.VMEM((GM, CHUNK), jnp.int32),
        pltpu.VMEM((GM * CHUNK, H), jnp.float32),
        pltpu.VMEM((GM * CHUNK, H), jnp.float32),
        pltpu.VMEM((GM * CHUNK,), jnp.float32),
        pltpu.VMEM((GM * CHUNK,), jnp.float32),
        pltpu.VMEM((CHUNK,), jnp.float32),
        pltpu.VMEM((SL + 16,), jnp.float32),
        pltpu.VMEM((SL, H), jnp.float32),
        pltpu.VMEM_SHARED((NPAD,), jnp.float32),
        pltpu.VMEM_SHARED((NPAD,), jnp.float32),
        pltpu.VMEM_SHARED((NPAD, H), jnp.float32),
        pltpu.VMEM_SHARED((NPAD, H), jnp.float32),
        pltpu.VMEM_SHARED((NPAD,), jnp.float32),
        pltpu.SemaphoreType.DMA,
        pltpu.SemaphoreType.DMA,
        pltpu.SemaphoreType.DMA,
        pltpu.SemaphoreType.DMA,
        pltpu.SemaphoreType.DMA,
        pltpu.SemaphoreType.DMA,
        pltpu.SemaphoreType.DMA,
    ],
    compiler_params=pltpu.CompilerParams(use_tc_tiling_on_sc=False,
                                         needs_layout_passes=False),
)
def _fused_kernel(row1_hbm, col1_hbm, xw_hbm, z1_hbm, z2_hbm,
                  acc_out, t_out, dinv_out, y_out,
                  dcol_v, ridx1_v, cidx1_v, d2a_v, d2b_v,
                  r2a_v, c2a_v, r2b_v, c2b_v, rowsa_v, rowsb_v, dva_v, dvb_v,
                  ones_v, dinvl_v, xwl_v,
                  deg_sh, dinv_sh, y_sh, acc_sh, t_sh,
                  semga, semgb, semsa, semsb, semdeg, semp, semq):
    c = lax.axis_index("c")
    s = lax.axis_index("s")
    w = c * NS + s

    cpd = pltpu.async_copy(
        col1_hbm.at[pl.ds(s * RD * CHUNK, RD * CHUNK)], dcol_v, semp)
    cpr = pltpu.async_copy(
        row1_hbm.at[pl.ds(w * RW * CHUNK, RW * CHUNK)], ridx1_v, semq)
    cpc = pltpu.async_copy(
        col1_hbm.at[pl.ds(w * RW * CHUNK, RW * CHUNK)], cidx1_v, semq)
    cpx = pltpu.async_copy(xw_hbm.at[pl.ds(s * SL, SL)], xwl_v, semq)

    for k in range(CHUNK // 16):
        ones_v[pl.ds(k * 16, 16)] = jnp.ones((16,), jnp.float32)
    pltpu.sync_copy(z1_hbm.at[pl.ds(s * SL, SL)], deg_sh.at[pl.ds(s * SL, SL)])
    pltpu.sync_copy(z1_hbm.at[pl.ds(s * SL, SL)], t_sh.at[pl.ds(s * SL, SL)])
    pltpu.sync_copy(z2_hbm.at[pl.ds(s * SL, SL)], acc_sh.at[pl.ds(s * SL, SL)])
    plsc.subcore_barrier()

    cpd.wait()

    def deg_outer(i, carry):
        cps = []
        for buf, g in ((d2a_v, 0), (d2b_v, 1)):
            for j in range(G):
                for k in range(CHUNK // 16):
                    o = ((i * 2 + g) * G + j) * CHUNK + k * 16
                    buf[j, pl.ds(k * 16, 16)] = dcol_v[pl.ds(o, 16)]
            for j in range(G):
                cps.append(pltpu.async_copy(ones_v, deg_sh.at[buf.at[j]],
                                            semdeg, add=True))
        for cp in cps:
            cp.wait()
        return carry

    lax.fori_loop(0, RD // (2 * G), deg_outer, 0)
    plsc.subcore_barrier()

    pltpu.sync_copy(deg_sh.at[pl.ds(s * SL, SL)], dinvl_v.at[pl.ds(0, SL)])
    cpx.wait()
    cpr.wait()
    cpc.wait()

    def rsq(k, carry):
        d = dinvl_v[pl.ds(k * 16, 16)] + 1.0
        bits = plsc.bitcast(d, jnp.int32)
        bits = 0x5F3759DF - lax.shift_right_logical(bits, 1)
        r = plsc.bitcast(bits, jnp.float32)
        r = r * (1.5 - 0.5 * d * r * r)
        r = r * (1.5 - 0.5 * d * r * r)
        r = r * (1.5 - 0.5 * d * r * r)
        dinvl_v[pl.ds(k * 16, 16)] = r
        return carry

    lax.fori_loop(0, SL // 16, rsq, 0)

    def scale_block(b, carry):
        r0 = b * 16
        dvv = dinvl_v[pl.ds(r0, 16)]
        for j in range(16):
            xwl_v[r0 + j] = xwl_v[r0 + j] * dvv[j]
        return carry

    lax.fori_loop(0, SL // 16, scale_block, 0)
    pltpu.sync_copy(dinvl_v.at[pl.ds(0, SL)], dinv_sh.at[pl.ds(s * SL, SL)])
    pltpu.sync_copy(xwl_v, y_sh.at[pl.ds(s * SL, SL)])

    @pl.when(c == 0)
    def _():
        pltpu.sync_copy(dinvl_v.at[pl.ds(0, SL)], dinv_out.at[pl.ds(s * SL, SL)])
        pltpu.sync_copy(xwl_v, y_out.at[pl.ds(s * SL, SL)])

    plsc.subcore_barrier()

    def _fire_gathers(base, rows_buf, dv_buf, semg):
        cpg = pltpu.async_copy(
            y_sh.at[ridx1_v.at[pl.ds(base, GM * CHUNK)]], rows_buf, semg)
        cpd2 = pltpu.async_copy(
            dinv_sh.at[cidx1_v.at[pl.ds(base, GM * CHUNK)]], dv_buf, semg)
        return cpg, cpd2

    def _copy_idx(base, r2_buf, c2_buf):
        for j in range(GM):
            for k in range(CHUNK // 16):
                o = base + j * CHUNK + k * 16
                c2_buf[j, pl.ds(k * 16, 16)] = cidx1_v[pl.ds(o, 16)]
                r2_buf[j, pl.ds(k * 16, 16)] = ridx1_v[pl.ds(o, 16)]

    def _fire_scatters(rows_buf, dv_buf, r2_buf, c2_buf, sems):
        cps = []
        for j in range(GM):
            cps.append(pltpu.async_copy(
                rows_buf.at[pl.ds(j * CHUNK, CHUNK)],
                acc_sh.at[c2_buf.at[j]], sems, add=True))
            cps.append(pltpu.async_copy(
                dv_buf.at[pl.ds(j * CHUNK, CHUNK)],
                t_sh.at[r2_buf.at[j]], sems, add=True))
        return cps

    def outer(i, carry):
        ba = (i * 2) * GM * CHUNK
        bb = ba + GM * CHUNK
        cpga, cpgda = _fire_gathers(ba, rowsa_v, dva_v, semga)
        _copy_idx(ba, r2a_v, c2a_v)
        cpga.wait()
        cpgda.wait()
        cpsa = _fire_scatters(rowsa_v, dva_v, r2a_v, c2a_v, semsa)
        cpgb, cpgdb = _fire_gathers(bb, rowsb_v, dvb_v, semgb)
        _copy_idx(bb, r2b_v, c2b_v)
        cpgb.wait()
        cpgdb.wait()
        cpsb = _fire_scatters(rowsb_v, dvb_v, r2b_v, c2b_v, semsb)
        for cp in cpsa:
            cp.wait()
        for cp in cpsb:
            cp.wait()
        return carry

    lax.fori_loop(0, RW // (2 * GM), outer, 0)
    plsc.subcore_barrier()

    pltpu.sync_copy(acc_sh.at[pl.ds(s * SL, SL)], acc_out.at[c, pl.ds(s * SL, SL)])
    pltpu.sync_copy(t_sh.at[pl.ds(s * SL, SL)],
                    t_out.at[pl.ds(c * NPAD + s * SL, SL)])


def _prep_body(ei_ref, pad_ref, x_ref, w1_ref, rowp_ref, colp_ref, xw_ref):
    ei = ei_ref[...]
    pad = pad_ref[...]
    rowp_ref[...] = jnp.concatenate([ei[0], pad])
    colp_ref[...] = jnp.concatenate([ei[1], pad])
    xw_ref[pl.ds(0, N), :] = jnp.dot(x_ref[...], w1_ref[...],
                                     preferred_element_type=jnp.float32)
    xw_ref[pl.ds(N, NPAD - N), :] = jnp.zeros((NPAD - N, H), jnp.float32)


_prep_call = pl.pallas_call(
    _prep_body,
    out_shape=(
        jax.ShapeDtypeStruct((EP,), jnp.int32),
        jax.ShapeDtypeStruct((EP,), jnp.int32),
        jax.ShapeDtypeStruct((NPAD, H), jnp.float32),
    ),
)


PK = NPAD // 8


def _final_body(accp_ref, y_ref, dinv_ref, s_ref, b1_ref, w2_ref, b2_ref,
                lw_ref, lb_ref, out_ref):
    acc = accp_ref[0] + accp_ref[1]
    h = jnp.maximum(dinv_ref[...] * (acc + y_ref[...]) + b1_ref[...], 0.0)
    q = s_ref[...] * h
    z128 = jnp.sum(q, axis=0, keepdims=True)
    z = z128[:, 0:H]
    for g in range(1, 8):
        z = z + z128[:, g * H:(g + 1) * H]
    pooled = jnp.dot(z * (1.0 / N), w2_ref[...],
                     preferred_element_type=jnp.float32) + b2_ref[...]
    out_ref[...] = jnp.dot(pooled, lw_ref[...],
                           preferred_element_type=jnp.float32) + lb_ref[...]


_final_call = pl.pallas_call(
    _final_body,
    out_shape=jax.ShapeDtypeStruct((1, OUT), jnp.float32),
)


def kernel(x, edge_index, W1, b1, W2, b2, lin_W, lin_b):
    npad_extra = NPAD - N
    pad_idx = (N + jnp.arange(EP - E, dtype=jnp.int32) % npad_extra)
    zeros1 = jnp.zeros((NPAD,), jnp.float32)
    zeros2 = jnp.zeros((NPAD, H), jnp.float32)

    rowp, colp, xw = _prep_call(edge_index, pad_idx, x, W1)
    accp, t1d, dinv, y = _fused_kernel(rowp, colp, xw, zeros1, zeros2)

    tp = t1d.reshape(NC, NPAD)
    t = tp[0] + tp[1]
    s = dinv * (dinv + t)
    s = jnp.where(jnp.arange(NPAD) < N, s, 0.0)
    s_exp = jnp.repeat(s, H).reshape(PK, 128)
    dinv_exp = jnp.repeat(dinv, H).reshape(PK, 128)
    accp2 = accp.reshape(NC, PK, 128)
    y2 = y.reshape(PK, 128)
    b1t = jnp.tile(b1, 8).reshape(1, 128)

    out = _final_call(accp2, y2, dinv_exp, s_exp, b1t,
                      W2, b2.reshape(1, H), lin_W, lin_b.reshape(1, OUT))
    return out.reshape(OUT)
